# initial kernel scaffold (unmeasured)
import numpy as np

import jax
import jax.numpy as jnp
from jax import lax
from jax.experimental import pallas as pl
from jax.experimental.pallas import tpu as pltpu

N_DEV = 32
SQ = 1024
D = 1024
HQ_LOC = 8
DH = 128
CHUNK = SQ // N_DEV
SCALE = 0.08838834764831843


def _rope_constants():
    inv = 1.0 / (10000.0 ** (np.arange(0, DH, 2) / DH))
    pos = np.arange(SQ)[:, None] * inv[None, :]
    cos = np.repeat(np.cos(pos), 2, axis=-1).astype(np.float32)
    sin = np.repeat(np.sin(pos), 2, axis=-1).astype(np.float32)
    perm = np.zeros((DH, DH), np.float32)
    for k in range(DH // 2):
        perm[2 * k + 1, 2 * k] = -1.0
        perm[2 * k, 2 * k + 1] = 1.0
    return cos, sin, perm


def kernel(x, Wq, Wk, Wv, Wo):
    cos_c, sin_c, perm_c = _rope_constants()

    def body(x_ref, wq_ref, wk_ref, wv_ref, wo_ref, cos_ref, sin_ref,
             perm_ref, out_ref, p_ref, comm_ref, send_sems, recv_sems):
        my = lax.axis_index("i")
        left = lax.rem(my + N_DEV - 1, N_DEV)
        right = lax.rem(my + 1, N_DEV)

        barrier_sem = pltpu.get_barrier_semaphore()
        for nbr in (left, right):
            pl.semaphore_signal(
                barrier_sem, inc=1,
                device_id=(nbr,), device_id_type=pl.DeviceIdType.MESH,
            )
        pl.semaphore_wait(barrier_sem, 2)

        xm = x_ref[0]
        q = jnp.dot(xm, wq_ref[...], preferred_element_type=jnp.float32)
        k = jnp.dot(xm, wk_ref[...], preferred_element_type=jnp.float32)
        v = jnp.dot(xm, wv_ref[...], preferred_element_type=jnp.float32)
        cos = cos_ref[...]
        sin = sin_ref[...]
        perm = perm_ref[...]

        ctx_parts = []
        for h in range(HQ_LOC):
            sl = slice(h * DH, (h + 1) * DH)
            qh = q[:, sl]
            kh = k[:, sl]
            vh = v[:, sl]
            qh = qh * cos + jnp.dot(qh, perm, preferred_element_type=jnp.float32) * sin
            kh = kh * cos + jnp.dot(kh, perm, preferred_element_type=jnp.float32) * sin
            s = lax.dot_general(
                qh, kh, (((1,), (1,)), ((), ())),
                preferred_element_type=jnp.float32,
            ) * SCALE
            m = jnp.max(s, axis=1, keepdims=True)
            e = jnp.exp(s - m)
            w = e / jnp.sum(e, axis=1, keepdims=True)
            ctx_parts.append(jnp.dot(w, vh, preferred_element_type=jnp.float32))
        ctx = jnp.concatenate(ctx_parts, axis=1)
        p_ref[...] = jnp.dot(ctx, wo_ref[...], preferred_element_type=jnp.float32)

        def cmod(c):
            return lax.rem(c + 4 * N_DEV * N_DEV, N_DEV)

        def crows(c):
            return pl.ds(c * CHUNK, CHUNK)

        comm_ref[0] = p_ref[crows(my), :]

        for hop in range(2 * (N_DEV - 1)):
            send_slot = hop % 2
            recv_slot = (hop + 1) % 2
            rdma = pltpu.make_async_remote_copy(
                src_ref=comm_ref.at[send_slot],
                dst_ref=comm_ref.at[recv_slot],
                send_sem=send_sems.at[send_slot],
                recv_sem=recv_sems.at[recv_slot],
                device_id=(right,),
                device_id_type=pl.DeviceIdType.MESH,
            )
            rdma.start()
            rdma.wait()

            if hop < N_DEV - 1:
                c = cmod(my - hop - 1)
                comm_ref[recv_slot] = comm_ref[recv_slot] + p_ref[crows(c), :]
                if hop == N_DEV - 2:
                    out_ref[0, crows(cmod(my + 1)), :] = comm_ref[recv_slot]
            else:
                s_ag = hop - (N_DEV - 1)
                c = cmod(my - s_ag)
                out_ref[0, crows(c), :] = comm_ref[recv_slot]

    grid_spec = pltpu.PrefetchScalarGridSpec(
        num_scalar_prefetch=0,
        in_specs=[pl.BlockSpec(memory_space=pltpu.VMEM)] * 8,
        out_specs=pl.BlockSpec(memory_space=pltpu.VMEM),
        scratch_shapes=[
            pltpu.VMEM((SQ, D), jnp.float32),
            pltpu.VMEM((2, CHUNK, D), jnp.float32),
            pltpu.SemaphoreType.DMA((2,)),
            pltpu.SemaphoreType.DMA((2,)),
        ],
    )
    return pl.pallas_call(
        body,
        grid_spec=grid_spec,
        out_shape=jax.ShapeDtypeStruct((1, SQ, D), jnp.float32),
        compiler_params=pltpu.CompilerParams(collective_id=0),
    )(x, Wq, Wk, Wv, Wo, jnp.asarray(cos_c), jnp.asarray(sin_c),
      jnp.asarray(perm_c))


# baseline (device time: 249730 ns/iter reference)
import numpy as np

import jax
import jax.numpy as jnp
from jax import lax
from jax.experimental import pallas as pl
from jax.experimental.pallas import tpu as pltpu

N_DEV = 32
SQ = 1024
D = 1024
HQ_LOC = 8
DH = 128
CHUNK = SQ // N_DEV
SCALE = 0.08838834764831843


def _rope_constants():
    inv = 1.0 / (10000.0 ** (np.arange(0, DH, 2) / DH))
    pos = np.arange(SQ)[:, None] * inv[None, :]
    cos = np.repeat(np.cos(pos), 2, axis=-1).astype(np.float32)
    sin = np.repeat(np.sin(pos), 2, axis=-1).astype(np.float32)
    perm = np.zeros((DH, DH), np.float32)
    for k in range(DH // 2):
        perm[2 * k + 1, 2 * k] = -1.0
        perm[2 * k, 2 * k + 1] = 1.0
    return cos, sin, perm


def kernel(x, Wq, Wk, Wv, Wo):
    cos_c, sin_c, perm_c = _rope_constants()

    def body(x_ref, wq_ref, wk_ref, wv_ref, wo_ref, cos_ref, sin_ref,
             perm_ref, out_ref, p_ref, comm_ref, send_sems, recv_sems):
        my = lax.axis_index("i")
        left = lax.rem(my + N_DEV - 1, N_DEV)
        right = lax.rem(my + 1, N_DEV)

        barrier_sem = pltpu.get_barrier_semaphore()
        for nbr in (left, right):
            pl.semaphore_signal(
                barrier_sem, inc=1,
                device_id=(nbr,), device_id_type=pl.DeviceIdType.MESH,
            )
        pl.semaphore_wait(barrier_sem, 2)

        xm = x_ref[0]
        q = jnp.dot(xm, wq_ref[...], preferred_element_type=jnp.float32)
        k = jnp.dot(xm, wk_ref[...], preferred_element_type=jnp.float32)
        v = jnp.dot(xm, wv_ref[...], preferred_element_type=jnp.float32)
        cos = cos_ref[...]
        sin = sin_ref[...]
        perm = perm_ref[...]

        ctx_parts = []
        for h in range(HQ_LOC):
            sl = slice(h * DH, (h + 1) * DH)
            qh = q[:, sl]
            kh = k[:, sl]
            vh = v[:, sl]
            qh = qh * cos + jnp.dot(qh, perm, preferred_element_type=jnp.float32) * sin
            kh = kh * cos + jnp.dot(kh, perm, preferred_element_type=jnp.float32) * sin
            s = lax.dot_general(
                qh, kh, (((1,), (1,)), ((), ())),
                preferred_element_type=jnp.float32,
            ) * SCALE
            m = jnp.max(s, axis=1, keepdims=True)
            e = jnp.exp(s - m)
            w = e / jnp.sum(e, axis=1, keepdims=True)
            ctx_parts.append(jnp.dot(w, vh, preferred_element_type=jnp.float32))
        ctx = jnp.concatenate(ctx_parts, axis=1)
        p_ref[...] = jnp.dot(ctx, wo_ref[...], preferred_element_type=jnp.float32)

        def cmod(c):
            return lax.rem(c + 4 * N_DEV * N_DEV, N_DEV)

        def crows(c):
            return pl.ds(c * CHUNK, CHUNK)

        comm_ref[0] = p_ref[crows(my), :]

        for hop in range(2 * (N_DEV - 1)):
            send_slot = hop % 2
            recv_slot = (hop + 1) % 2
            rdma = pltpu.make_async_remote_copy(
                src_ref=comm_ref.at[send_slot],
                dst_ref=comm_ref.at[recv_slot],
                send_sem=send_sems.at[send_slot],
                recv_sem=recv_sems.at[recv_slot],
                device_id=(right,),
                device_id_type=pl.DeviceIdType.MESH,
            )
            rdma.start()
            rdma.wait()

            if hop < N_DEV - 1:
                c = cmod(my - hop - 1)
                comm_ref[recv_slot] = comm_ref[recv_slot] + p_ref[crows(c), :]
                if hop == N_DEV - 2:
                    out_ref[0, crows(cmod(my + 1)), :] = comm_ref[recv_slot]
            else:
                s_ag = hop - (N_DEV - 1)
                c = cmod(my - s_ag)
                out_ref[0, crows(c), :] = comm_ref[recv_slot]

    return pl.pallas_call(
        body,
        out_shape=jax.ShapeDtypeStruct((1, SQ, D), jnp.float32),
        in_specs=[pl.BlockSpec(memory_space=pltpu.VMEM)] * 8,
        out_specs=pl.BlockSpec(memory_space=pltpu.VMEM),
        scratch_shapes=[
            pltpu.VMEM((SQ, D), jnp.float32),
            pltpu.VMEM((2, CHUNK, D), jnp.float32),
            pltpu.SemaphoreType.DMA((2,)),
            pltpu.SemaphoreType.DMA((2,)),
        ],
        compiler_params=pltpu.CompilerParams(collective_id=0),
    )(x, Wq, Wk, Wv, Wo, jnp.asarray(cos_c), jnp.asarray(sin_c),
      jnp.asarray(perm_c))


# device time: 172754 ns/iter; 1.4456x vs baseline; 1.4456x over previous
import numpy as np

import jax
import jax.numpy as jnp
from jax import lax
from jax.experimental import pallas as pl
from jax.experimental.pallas import tpu as pltpu

N_DEV = 32
NZ = 4
NP = 8
SQ = 1024
D = 1024
HQ_LOC = 8
DH = 128
C8 = SQ // NP
C4 = C8 // NZ
SCALE = 0.08838834764831843

RING_J = (0, 3, 4, 7, 6, 5, 2, 1)
J_TO_R = (0, 7, 6, 1, 2, 5, 4, 3)


def _rope_constants():
    inv = 1.0 / (10000.0 ** (np.arange(0, DH, 2) / DH))
    pos = np.arange(SQ)[:, None] * inv[None, :]
    cos = np.repeat(np.cos(pos), 2, axis=-1).astype(np.float32)
    sin = np.repeat(np.sin(pos), 2, axis=-1).astype(np.float32)
    perm = np.zeros((DH, DH), np.float32)
    for k in range(DH // 2):
        perm[2 * k + 1, 2 * k] = -1.0
        perm[2 * k, 2 * k + 1] = 1.0
    return cos, sin, perm


def _lookup(idx, table):
    out = jnp.int32(table[0])
    for v in range(1, len(table)):
        out = jnp.where(idx == v, jnp.int32(table[v]), out)
    return out


def kernel(x, Wq, Wk, Wv, Wo):
    cos_c, sin_c, perm_c = _rope_constants()
    bf16 = jnp.bfloat16
    f32 = jnp.float32

    def body(x_ref, wq_ref, wk_ref, wv_ref, wo_ref, cos_ref, sin_ref,
             perm_ref, out_ref, p_ref, comm8_ref, commz_ref, agc_ref,
             send8, recv8, sendz, recvz, sendag, recvag):
        my = lax.axis_index("i")
        z = lax.div(my, NP)
        j = lax.rem(my, NP)
        r = _lookup(j, J_TO_R)

        def plane_dev(rr):
            return z * NP + _lookup(rr, RING_J)

        p_right = plane_dev(lax.rem(r + 1, NP))
        p_left = plane_dev(lax.rem(r + NP - 1, NP))
        z_right = lax.rem(z + 1, NZ) * NP + j
        z_left = lax.rem(z + NZ - 1, NZ) * NP + j

        barrier_sem = pltpu.get_barrier_semaphore()
        for nbr in (p_left, p_right, z_left, z_right):
            pl.semaphore_signal(
                barrier_sem, inc=1,
                device_id=(nbr,), device_id_type=pl.DeviceIdType.MESH,
            )
        pl.semaphore_wait(barrier_sem, 4)

        xb = x_ref[0].astype(bf16)
        q = jnp.dot(xb, wq_ref[...].astype(bf16), preferred_element_type=f32)
        k = jnp.dot(xb, wk_ref[...].astype(bf16), preferred_element_type=f32)
        v = jnp.dot(xb, wv_ref[...].astype(bf16), preferred_element_type=f32)
        cos = cos_ref[...]
        sin = sin_ref[...]
        permb = perm_ref[...].astype(bf16)

        ctx_parts = []
        for h in range(HQ_LOC):
            sl = slice(h * DH, (h + 1) * DH)
            qh = q[:, sl]
            kh = k[:, sl]
            vh = v[:, sl]
            qh = qh * cos + jnp.dot(
                qh.astype(bf16), permb, preferred_element_type=f32) * sin
            kh = kh * cos + jnp.dot(
                kh.astype(bf16), permb, preferred_element_type=f32) * sin
            s = lax.dot_general(
                qh.astype(bf16), kh.astype(bf16), (((1,), (1,)), ((), ())),
                preferred_element_type=f32,
            ) * SCALE
            m = jnp.max(s, axis=1, keepdims=True)
            e = jnp.exp(s - m)
            w = e / jnp.sum(e, axis=1, keepdims=True)
            ctx_parts.append(jnp.dot(
                w.astype(bf16), vh.astype(bf16), preferred_element_type=f32))
        ctx = jnp.concatenate(ctx_parts, axis=1)
        p_ref[...] = jnp.dot(
            ctx.astype(bf16), wo_ref[...].astype(bf16),
            preferred_element_type=f32)

        def ring_hop(comm, ssem, rsem, slot_s, slot_r, target):
            rdma = pltpu.make_async_remote_copy(
                src_ref=comm.at[slot_s],
                dst_ref=comm.at[slot_r],
                send_sem=ssem.at[slot_s],
                recv_sem=rsem.at[slot_r],
                device_id=(target,),
                device_id_type=pl.DeviceIdType.MESH,
            )
            rdma.start()
            rdma.wait()

        def mod(val, n):
            return lax.rem(val + 8 * n * n, n)

        def rows8(c):
            return pl.ds(c * C8, C8)

        comm8_ref[0] = p_ref[rows8(r), :]
        for hop in range(NP - 1):
            ss, rs = hop % 2, (hop + 1) % 2
            ring_hop(comm8_ref, send8, recv8, ss, rs, p_right)
            c = mod(r - hop - 1, NP)
            comm8_ref[rs] = comm8_ref[rs] + p_ref[rows8(c), :]
        agc_ref[0] = comm8_ref[1]
        c1 = mod(r + 1, NP)

        def sub(kk):
            return pl.ds(kk * C4, C4)

        def out_rows(kk):
            return pl.ds(c1 * C8 + kk * C4, C4)

        commz_ref[0] = agc_ref[0, sub(z), :]
        for hop in range(NZ - 1):
            ss, rs = hop % 2, (hop + 1) % 2
            ring_hop(commz_ref, sendz, recvz, ss, rs, z_right)
            c = mod(z - hop - 1, NZ)
            commz_ref[rs] = commz_ref[rs] + agc_ref[0, sub(c), :]
        kown = mod(z + 1, NZ)
        agc_ref[0, sub(kown), :] = commz_ref[1]
        out_ref[0, out_rows(kown), :] = commz_ref[1]

        for hop in range(NZ - 1, 2 * (NZ - 1)):
            ss, rs = hop % 2, (hop + 1) % 2
            ring_hop(commz_ref, sendz, recvz, ss, rs, z_right)
            c = mod(z - (hop - (NZ - 1)), NZ)
            agc_ref[0, sub(c), :] = commz_ref[rs]
            out_ref[0, out_rows(c), :] = commz_ref[rs]

        for a in range(NP - 1):
            ss, rs = a % 2, (a + 1) % 2
            ring_hop(agc_ref, sendag, recvag, ss, rs, p_right)
            c = mod(r - a, NP)
            out_ref[0, rows8(c), :] = agc_ref[rs]

    return pl.pallas_call(
        body,
        out_shape=jax.ShapeDtypeStruct((1, SQ, D), f32),
        in_specs=[pl.BlockSpec(memory_space=pltpu.VMEM)] * 8,
        out_specs=pl.BlockSpec(memory_space=pltpu.VMEM),
        scratch_shapes=[
            pltpu.VMEM((SQ, D), f32),
            pltpu.VMEM((2, C8, D), f32),
            pltpu.VMEM((2, C4, D), f32),
            pltpu.VMEM((2, C8, D), f32),
            pltpu.SemaphoreType.DMA((2,)),
            pltpu.SemaphoreType.DMA((2,)),
            pltpu.SemaphoreType.DMA((2,)),
            pltpu.SemaphoreType.DMA((2,)),
            pltpu.SemaphoreType.DMA((2,)),
            pltpu.SemaphoreType.DMA((2,)),
        ],
        compiler_params=pltpu.CompilerParams(collective_id=0),
    )(x, Wq, Wk, Wv, Wo, jnp.asarray(cos_c), jnp.asarray(sin_c),
      jnp.asarray(perm_c))


# device time: 134581 ns/iter; 1.8556x vs baseline; 1.2836x over previous
import numpy as np

import jax
import jax.numpy as jnp
from jax import lax
from jax.experimental import pallas as pl
from jax.experimental.pallas import tpu as pltpu

N_DEV = 32
NZ = 4
NP = 8
SQ = 1024
D = 1024
HQ_LOC = 8
DH = 128
C8 = SQ // NP
HALF = C8 // 2
C4 = C8 // NZ
SCALE = 0.08838834764831843

RING_J = (0, 3, 4, 7, 6, 5, 2, 1)
J_TO_R = (0, 7, 6, 1, 2, 5, 4, 3)


def _rope_constants():
    inv = 1.0 / (10000.0 ** (np.arange(0, DH, 2) / DH))
    pos = np.arange(SQ)[:, None] * inv[None, :]
    cos = np.repeat(np.cos(pos), 2, axis=-1).astype(np.float32)
    sin = np.repeat(np.sin(pos), 2, axis=-1).astype(np.float32)
    perm = np.zeros((DH, DH), np.float32)
    for k in range(DH // 2):
        perm[2 * k + 1, 2 * k] = -1.0
        perm[2 * k, 2 * k + 1] = 1.0
    return cos, sin, perm


def _lookup(idx, table):
    out = jnp.int32(table[0])
    for v in range(1, len(table)):
        out = jnp.where(idx == v, jnp.int32(table[v]), out)
    return out


def kernel(x, Wq, Wk, Wv, Wo):
    cos_c, sin_c, perm_c = _rope_constants()
    bf16 = jnp.bfloat16
    f32 = jnp.float32

    def body(x_ref, wq_ref, wk_ref, wv_ref, wo_ref, cos_ref, sin_ref,
             perm_ref, out_ref, p_ref, cw_ref, ccw_ref, zasm_ref, commz_ref,
             agcw_ref, agccw_ref, sendcw, recvcw, sendccw, recvccw,
             sendz, recvz, sendagcw, recvagcw, sendagccw, recvagccw):
        my = lax.axis_index("i")
        z = lax.div(my, NP)
        j = lax.rem(my, NP)
        r = _lookup(j, J_TO_R)

        def plane_dev(rr):
            return z * NP + _lookup(rr, RING_J)

        p_right = plane_dev(lax.rem(r + 1, NP))
        p_left = plane_dev(lax.rem(r + NP - 1, NP))
        z_right = lax.rem(z + 1, NZ) * NP + j
        z_left = lax.rem(z + NZ - 1, NZ) * NP + j

        barrier_sem = pltpu.get_barrier_semaphore()
        for nbr in (p_left, p_right, z_left, z_right):
            pl.semaphore_signal(
                barrier_sem, inc=1,
                device_id=(nbr,), device_id_type=pl.DeviceIdType.MESH,
            )
        pl.semaphore_wait(barrier_sem, 4)

        xb = x_ref[0].astype(bf16)
        q = jnp.dot(xb, wq_ref[...].astype(bf16), preferred_element_type=f32)
        k = jnp.dot(xb, wk_ref[...].astype(bf16), preferred_element_type=f32)
        v = jnp.dot(xb, wv_ref[...].astype(bf16), preferred_element_type=f32)
        cos = cos_ref[...]
        sin = sin_ref[...]
        permb = perm_ref[...].astype(bf16)

        ctx_parts = []
        for h in range(HQ_LOC):
            sl = slice(h * DH, (h + 1) * DH)
            qh = q[:, sl]
            kh = k[:, sl]
            vh = v[:, sl]
            qh = qh * cos + jnp.dot(
                qh.astype(bf16), permb, preferred_element_type=f32) * sin
            kh = kh * cos + jnp.dot(
                kh.astype(bf16), permb, preferred_element_type=f32) * sin
            s = lax.dot_general(
                qh.astype(bf16), kh.astype(bf16), (((1,), (1,)), ((), ())),
                preferred_element_type=f32,
            ) * SCALE
            m = jnp.max(s, axis=1, keepdims=True)
            e = jnp.exp(s - m)
            w = e / jnp.sum(e, axis=1, keepdims=True)
            ctx_parts.append(jnp.dot(
                w.astype(bf16), vh.astype(bf16), preferred_element_type=f32))
        ctx = jnp.concatenate(ctx_parts, axis=1)
        p_ref[...] = jnp.dot(
            ctx.astype(bf16), wo_ref[...].astype(bf16),
            preferred_element_type=f32)

        def hop(pairs, slot_s, slot_r):
            rdmas = []
            for comm, ssem, rsem, target in pairs:
                rdma = pltpu.make_async_remote_copy(
                    src_ref=comm.at[slot_s],
                    dst_ref=comm.at[slot_r],
                    send_sem=ssem.at[slot_s],
                    recv_sem=rsem.at[slot_r],
                    device_id=(target,),
                    device_id_type=pl.DeviceIdType.MESH,
                )
                rdma.start()
                rdmas.append(rdma)
            for rdma in rdmas:
                rdma.wait()

        def mod(val, n):
            return lax.rem(val + 8 * n * n, n)

        def top(c):
            return pl.ds(c * C8, HALF)

        def bot(c):
            return pl.ds(c * C8 + HALF, HALF)

        cw_pair = (cw_ref, sendcw, recvcw, p_right)
        ccw_pair = (ccw_ref, sendccw, recvccw, p_left)

        cw_ref[0] = p_ref[top(r), :]
        ccw_ref[0] = p_ref[bot(r), :]
        for hh in range(NP - 1):
            ss, rs = hh % 2, (hh + 1) % 2
            hop([cw_pair, ccw_pair], ss, rs)
            cw_ref[rs] = cw_ref[rs] + p_ref[top(mod(r - hh - 1, NP)), :]
            ccw_ref[rs] = ccw_ref[rs] + p_ref[bot(mod(r + hh + 1, NP)), :]
        zasm_ref[0:HALF] = cw_ref[1]
        zasm_ref[HALF:C8] = ccw_ref[1]
        c1t = mod(r + 1, NP)
        c1b = mod(r + NP - 1, NP)

        def zsub(kk):
            return pl.ds(kk * C4, C4)

        def out_rows(kk):
            st = jnp.where(
                kk < 2,
                c1t * C8 + kk * C4,
                c1b * C8 + HALF + (kk - 2) * C4,
            )
            return pl.ds(st, C4)

        z_pair = (commz_ref, sendz, recvz, z_right)

        commz_ref[0] = zasm_ref[zsub(z), :]
        for hh in range(NZ - 1):
            ss, rs = hh % 2, (hh + 1) % 2
            hop([z_pair], ss, rs)
            commz_ref[rs] = commz_ref[rs] + zasm_ref[zsub(mod(z - hh - 1, NZ)), :]
        kown = mod(z + 1, NZ)
        zasm_ref[zsub(kown), :] = commz_ref[1]
        out_ref[0, out_rows(kown), :] = commz_ref[1]

        for hh in range(NZ - 1, 2 * (NZ - 1)):
            ss, rs = hh % 2, (hh + 1) % 2
            hop([z_pair], ss, rs)
            c = mod(z - (hh - (NZ - 1)), NZ)
            zasm_ref[zsub(c), :] = commz_ref[rs]
            out_ref[0, out_rows(c), :] = commz_ref[rs]

        agcw_ref[0] = zasm_ref[0:HALF]
        agccw_ref[0] = zasm_ref[HALF:C8]
        agcw_pair = (agcw_ref, sendagcw, recvagcw, p_right)
        agccw_pair = (agccw_ref, sendagccw, recvagccw, p_left)
        for a in range(NP - 1):
            ss, rs = a % 2, (a + 1) % 2
            hop([agcw_pair, agccw_pair], ss, rs)
            out_ref[0, top(mod(r - a, NP)), :] = agcw_ref[rs]
            out_ref[0, bot(mod(r + a, NP)), :] = agccw_ref[rs]

    return pl.pallas_call(
        body,
        out_shape=jax.ShapeDtypeStruct((1, SQ, D), f32),
        in_specs=[pl.BlockSpec(memory_space=pltpu.VMEM)] * 8,
        out_specs=pl.BlockSpec(memory_space=pltpu.VMEM),
        scratch_shapes=[
            pltpu.VMEM((SQ, D), f32),
            pltpu.VMEM((2, HALF, D), f32),
            pltpu.VMEM((2, HALF, D), f32),
            pltpu.VMEM((C8, D), f32),
            pltpu.VMEM((2, C4, D), f32),
            pltpu.VMEM((2, HALF, D), f32),
            pltpu.VMEM((2, HALF, D), f32),
            pltpu.SemaphoreType.DMA((2,)),
            pltpu.SemaphoreType.DMA((2,)),
            pltpu.SemaphoreType.DMA((2,)),
            pltpu.SemaphoreType.DMA((2,)),
            pltpu.SemaphoreType.DMA((2,)),
            pltpu.SemaphoreType.DMA((2,)),
            pltpu.SemaphoreType.DMA((2,)),
            pltpu.SemaphoreType.DMA((2,)),
            pltpu.SemaphoreType.DMA((2,)),
            pltpu.SemaphoreType.DMA((2,)),
        ],
        compiler_params=pltpu.CompilerParams(collective_id=0),
    )(x, Wq, Wk, Wv, Wo, jnp.asarray(cos_c), jnp.asarray(sin_c),
      jnp.asarray(perm_c))


# device time: 132821 ns/iter; 1.8802x vs baseline; 1.0133x over previous
import numpy as np

import jax
import jax.numpy as jnp
from jax import lax
from jax.experimental import pallas as pl
from jax.experimental.pallas import tpu as pltpu

N_DEV = 32
NZ = 4
NP = 8
SQ = 1024
D = 1024
HQ_LOC = 8
DH = 128
C8 = SQ // NP
HALF = C8 // 2
C4 = C8 // NZ
SCALE = 0.08838834764831843

RING_J = (0, 3, 4, 7, 6, 5, 2, 1)
J_TO_R = (0, 7, 6, 1, 2, 5, 4, 3)


def _rope_constants():
    inv = 1.0 / (10000.0 ** (np.arange(0, DH, 2) / DH))
    pos = np.arange(SQ)[:, None] * inv[None, :]
    cos = np.repeat(np.cos(pos), 2, axis=-1).astype(np.float32)
    sin = np.repeat(np.sin(pos), 2, axis=-1).astype(np.float32)
    perm = np.zeros((DH, DH), np.float32)
    for k in range(DH // 2):
        perm[2 * k + 1, 2 * k] = -1.0
        perm[2 * k, 2 * k + 1] = 1.0
    return cos, sin, perm


def _lookup(idx, table):
    out = jnp.int32(table[0])
    for v in range(1, len(table)):
        out = jnp.where(idx == v, jnp.int32(table[v]), out)
    return out


def kernel(x, Wq, Wk, Wv, Wo):
    cos_c, sin_c, perm_c = _rope_constants()
    bf16 = jnp.bfloat16
    f32 = jnp.float32

    def body(x_ref, wq_ref, wk_ref, wv_ref, wo_ref, cos_ref, sin_ref,
             perm_ref, out_ref, p_ref, q_ref, k_ref, v_ref, wob_ref,
             cw_ref, ccw_ref, zasm_ref, commz_ref, agcw_ref, agccw_ref,
             sendcw, recvcw, sendccw, recvccw, sendz, recvz,
             sendagcw, recvagcw, sendagccw, recvagccw):
        my = lax.axis_index("i")
        z = lax.div(my, NP)
        j = lax.rem(my, NP)
        r = _lookup(j, J_TO_R)

        def plane_dev(rr):
            return z * NP + _lookup(rr, RING_J)

        p_right = plane_dev(lax.rem(r + 1, NP))
        p_left = plane_dev(lax.rem(r + NP - 1, NP))
        z_right = lax.rem(z + 1, NZ) * NP + j
        z_left = lax.rem(z + NZ - 1, NZ) * NP + j

        barrier_sem = pltpu.get_barrier_semaphore()
        for nbr in (p_left, p_right, z_left, z_right):
            pl.semaphore_signal(
                barrier_sem, inc=1,
                device_id=(nbr,), device_id_type=pl.DeviceIdType.MESH,
            )
        pl.semaphore_wait(barrier_sem, 4)

        xb = x_ref[0].astype(bf16)
        q = jnp.dot(xb, wq_ref[...].astype(bf16), preferred_element_type=f32)
        k = jnp.dot(xb, wk_ref[...].astype(bf16), preferred_element_type=f32)
        v = jnp.dot(xb, wv_ref[...].astype(bf16), preferred_element_type=f32)
        cos = cos_ref[...]
        sin = sin_ref[...]
        permb = perm_ref[...].astype(bf16)
        for h in range(HQ_LOC):
            sl = slice(h * DH, (h + 1) * DH)
            qh = q[:, sl]
            kh = k[:, sl]
            qh = qh * cos + jnp.dot(
                qh.astype(bf16), permb, preferred_element_type=f32) * sin
            kh = kh * cos + jnp.dot(
                kh.astype(bf16), permb, preferred_element_type=f32) * sin
            q_ref[:, sl] = qh.astype(bf16)
            k_ref[:, sl] = kh.astype(bf16)
        v_ref[...] = v.astype(bf16)
        wob_ref[...] = wo_ref[...].astype(bf16)

        def compute_chunk(c):
            st = c * C8
            rows = pl.ds(st, C8)
            ctx_parts = []
            for h in range(HQ_LOC):
                sl = slice(h * DH, (h + 1) * DH)
                qhb = q_ref[rows, sl]
                s = lax.dot_general(
                    qhb, k_ref[:, sl], (((1,), (1,)), ((), ())),
                    preferred_element_type=f32,
                ) * SCALE
                m = jnp.max(s, axis=1, keepdims=True)
                e = jnp.exp(s - m)
                w = e / jnp.sum(e, axis=1, keepdims=True)
                ctx_parts.append(jnp.dot(
                    w.astype(bf16), v_ref[:, sl],
                    preferred_element_type=f32).astype(bf16))
            ctxb = jnp.concatenate(ctx_parts, axis=1)
            p_ref[rows, :] = jnp.dot(
                ctxb, wob_ref[...], preferred_element_type=f32)

        def start_hop(pairs, slot_s, slot_r):
            rdmas = []
            for comm, ssem, rsem, target in pairs:
                rdma = pltpu.make_async_remote_copy(
                    src_ref=comm.at[slot_s],
                    dst_ref=comm.at[slot_r],
                    send_sem=ssem.at[slot_s],
                    recv_sem=rsem.at[slot_r],
                    device_id=(target,),
                    device_id_type=pl.DeviceIdType.MESH,
                )
                rdma.start()
                rdmas.append(rdma)
            return rdmas

        def hop(pairs, slot_s, slot_r):
            for rdma in start_hop(pairs, slot_s, slot_r):
                rdma.wait()

        def mod(val, n):
            return lax.rem(val + 8 * n * n, n)

        def top(c):
            return pl.ds(c * C8, HALF)

        def bot(c):
            return pl.ds(c * C8 + HALF, HALF)

        cw_pair = (cw_ref, sendcw, recvcw, p_right)
        ccw_pair = (ccw_ref, sendccw, recvccw, p_left)

        compute_chunk(r)
        cw_ref[0] = p_ref[top(r), :]
        ccw_ref[0] = p_ref[bot(r), :]
        for hh in range(NP - 1):
            ss, rs = hh % 2, (hh + 1) % 2
            rdmas = start_hop([cw_pair, ccw_pair], ss, rs)
            if hh < 3:
                compute_chunk(mod(r - hh - 1, NP))
                compute_chunk(mod(r + hh + 1, NP))
            elif hh == 3:
                compute_chunk(mod(r + 4, NP))
            for rdma in rdmas:
                rdma.wait()
            cw_ref[rs] = cw_ref[rs] + p_ref[top(mod(r - hh - 1, NP)), :]
            ccw_ref[rs] = ccw_ref[rs] + p_ref[bot(mod(r + hh + 1, NP)), :]
        zasm_ref[0:HALF] = cw_ref[1]
        zasm_ref[HALF:C8] = ccw_ref[1]
        c1t = mod(r + 1, NP)
        c1b = mod(r + NP - 1, NP)

        def zsub(kk):
            return pl.ds(kk * C4, C4)

        def out_rows(kk):
            st = jnp.where(
                kk < 2,
                c1t * C8 + kk * C4,
                c1b * C8 + HALF + (kk - 2) * C4,
            )
            return pl.ds(st, C4)

        z_pair = (commz_ref, sendz, recvz, z_right)

        commz_ref[0] = zasm_ref[zsub(z), :]
        for hh in range(NZ - 1):
            ss, rs = hh % 2, (hh + 1) % 2
            hop([z_pair], ss, rs)
            commz_ref[rs] = commz_ref[rs] + zasm_ref[zsub(mod(z - hh - 1, NZ)), :]
        kown = mod(z + 1, NZ)
        zasm_ref[zsub(kown), :] = commz_ref[1]
        out_ref[0, out_rows(kown), :] = commz_ref[1]

        for hh in range(NZ - 1, 2 * (NZ - 1)):
            ss, rs = hh % 2, (hh + 1) % 2
            hop([z_pair], ss, rs)
            c = mod(z - (hh - (NZ - 1)), NZ)
            zasm_ref[zsub(c), :] = commz_ref[rs]
            out_ref[0, out_rows(c), :] = commz_ref[rs]

        agcw_ref[0] = zasm_ref[0:HALF]
        agccw_ref[0] = zasm_ref[HALF:C8]
        agcw_pair = (agcw_ref, sendagcw, recvagcw, p_right)
        agccw_pair = (agccw_ref, sendagccw, recvagccw, p_left)
        for a in range(NP - 1):
            ss, rs = a % 2, (a + 1) % 2
            hop([agcw_pair, agccw_pair], ss, rs)
            out_ref[0, top(mod(r - a, NP)), :] = agcw_ref[rs]
            out_ref[0, bot(mod(r + a, NP)), :] = agccw_ref[rs]

    return pl.pallas_call(
        body,
        out_shape=jax.ShapeDtypeStruct((1, SQ, D), f32),
        in_specs=[pl.BlockSpec(memory_space=pltpu.VMEM)] * 8,
        out_specs=pl.BlockSpec(memory_space=pltpu.VMEM),
        scratch_shapes=[
            pltpu.VMEM((SQ, D), f32),
            pltpu.VMEM((SQ, D), bf16),
            pltpu.VMEM((SQ, D), bf16),
            pltpu.VMEM((SQ, D), bf16),
            pltpu.VMEM((D, D), bf16),
            pltpu.VMEM((2, HALF, D), f32),
            pltpu.VMEM((2, HALF, D), f32),
            pltpu.VMEM((C8, D), f32),
            pltpu.VMEM((2, C4, D), f32),
            pltpu.VMEM((2, HALF, D), f32),
            pltpu.VMEM((2, HALF, D), f32),
            pltpu.SemaphoreType.DMA((2,)),
            pltpu.SemaphoreType.DMA((2,)),
            pltpu.SemaphoreType.DMA((2,)),
            pltpu.SemaphoreType.DMA((2,)),
            pltpu.SemaphoreType.DMA((2,)),
            pltpu.SemaphoreType.DMA((2,)),
            pltpu.SemaphoreType.DMA((2,)),
            pltpu.SemaphoreType.DMA((2,)),
            pltpu.SemaphoreType.DMA((2,)),
            pltpu.SemaphoreType.DMA((2,)),
        ],
        compiler_params=pltpu.CompilerParams(collective_id=0),
    )(x, Wq, Wk, Wv, Wo, jnp.asarray(cos_c), jnp.asarray(sin_c),
      jnp.asarray(perm_c))
